# baseline (speedup 1.0000x reference)
import functools

import jax
import jax.numpy as jnp
from jax import lax
from jax.experimental import pallas as pl
from jax.experimental.pallas import tpu as pltpu
from jax.experimental.pallas import tpu_sc as plsc

_MAJORITY_CLASS = 0
_NUM_CLASSES = 2


@functools.lru_cache(maxsize=None)
def _build_fill(batch: int):
    info = plsc.get_sparse_core_info()
    nc, ns, lanes = info.num_cores, info.num_subcores, info.num_lanes
    nw = nc * ns
    total = batch * _NUM_CLASSES
    assert total % nw == 0
    chunk = total // nw
    assert chunk % lanes == 0 and chunk % 8 == 0

    mesh = plsc.VectorSubcoreMesh(core_axis_name="c", subcore_axis_name="s")

    @functools.partial(
        pl.kernel,
        mesh=mesh,
        out_type=jax.ShapeDtypeStruct((total,), jnp.float32),
        scratch_types=[pltpu.VMEM((chunk,), jnp.float32)],
    )
    def fill(out_hbm, buf):
        wid = lax.axis_index("s") * nc + lax.axis_index("c")
        base = wid * chunk
        lane = lax.iota(jnp.int32, lanes)
        pattern = jnp.where(
            lax.rem(lane, _NUM_CLASSES) == _MAJORITY_CLASS,
            jnp.float32(0.0),
            jnp.float32(-jnp.inf),
        )
        for i in range(chunk // lanes):
            buf[pl.ds(i * lanes, lanes)] = pattern
        pltpu.sync_copy(buf, out_hbm.at[pl.ds(base, chunk)])

    @jax.jit
    def run():
        return fill().reshape(batch, _NUM_CLASSES)

    return run


def kernel(x):
    return _build_fill(x.shape[0])()


# speedup vs baseline: 2.0575x; 2.0575x over previous
import functools

import jax
import jax.numpy as jnp
from jax import lax
from jax.experimental import pallas as pl

_MAJORITY_CLASS = 0
_NUM_CLASSES = 2


def _fill_body(out_ref):
    lane = lax.broadcasted_iota(jnp.int32, out_ref.shape, 1)
    out_ref[...] = jnp.where(
        lax.rem(lane, _NUM_CLASSES) == _MAJORITY_CLASS,
        jnp.float32(0.0),
        jnp.float32(-jnp.inf),
    )


@functools.lru_cache(maxsize=None)
def _build(batch: int):
    total = batch * _NUM_CLASSES
    rows = total // 128

    @jax.jit
    def run():
        flat2d = pl.pallas_call(
            _fill_body,
            out_shape=jax.ShapeDtypeStruct((rows, 128), jnp.float32),
        )()
        return flat2d.reshape(batch, _NUM_CLASSES)

    return run


def kernel(x):
    return _build(x.shape[0])()
